# baseline (speedup 1.0000x reference)
import functools

import jax
import jax.numpy as jnp
from jax import lax
from jax.experimental import pallas as pl
from jax.experimental.pallas import tpu as pltpu
from jax.experimental.pallas import tpu_sc as plsc

VOCAB = 1000000
C = VOCAB - 1
DIM = 64
MODEL_DIM = 1024

NC, NS, LANES = 2, 16, 16
NW = NC * NS

CHUNK = 128


def _gather_body(seq_len, bpw, tok_hbm, table_hbm, out_hbm, tok_v, idx_v,
                 rows_v, sem):
    wid = lax.axis_index("s") * NC + lax.axis_index("c")
    base = wid * bpw
    pltpu.sync_copy(tok_hbm.at[pl.ds(base, bpw + 8)], tok_v)
    lane = lax.iota(jnp.int32, LANES)
    nchunk = bpw // CHUNK
    for j in range(bpw // LANES):
        cur = tok_v[pl.ds(8 + LANES * j, LANES)]
        prev = tok_v[pl.ds(7 + LANES * j, LANES)]
        h = jnp.mod(
            jnp.bitwise_xor(jnp.int32(36313) * cur, jnp.int32(27191) * prev),
            jnp.int32(C))
        pos = base + (LANES * j) + lane
        h = jnp.where((pos & (seq_len - 1)) == 0, jnp.int32(C), h)
        idx_v[j // (CHUNK // LANES), pl.ds((j % (CHUNK // LANES)) * LANES,
                                           LANES)] = h
    copies = [
        pltpu.async_copy(table_hbm.at[idx_v.at[i]], rows_v.at[i], sem)
        for i in range(nchunk)
    ]
    for cp in copies:
        cp.wait()
    pltpu.sync_copy(rows_v, out_hbm.at[pl.ds(wid * nchunk, nchunk)])


def _proj_body(e_ref, p_ref, o_ref):
    o_ref[...] = lax.dot_general(
        e_ref[...], p_ref[...], (((1,), (1,)), ((), ())),
        preferred_element_type=jnp.float32)


@jax.jit
def kernel(token_ids, embed_weight, proj_weight, scale):
    batch, seq_len = token_ids.shape
    tok = batch * seq_len
    bpw = tok // NW
    nchunk = bpw // CHUNK

    tok_pad = jnp.concatenate(
        [jnp.zeros((8,), jnp.int32),
         token_ids.reshape(-1)])

    mesh = plsc.VectorSubcoreMesh(core_axis_name="c", subcore_axis_name="s")
    gathered = pl.kernel(
        functools.partial(_gather_body, seq_len, bpw),
        out_type=jax.ShapeDtypeStruct((tok // CHUNK, CHUNK, DIM),
                                      jnp.float32),
        mesh=mesh,
        scratch_types=[
            pltpu.VMEM((bpw + 8,), jnp.int32),
            pltpu.VMEM((nchunk, CHUNK), jnp.int32),
            pltpu.VMEM((nchunk, CHUNK, DIM), jnp.float32),
            pltpu.SemaphoreType.DMA,
        ],
        compiler_params=pltpu.CompilerParams(use_tc_tiling_on_sc=False),
    )(tok_pad, embed_weight)

    e = gathered.reshape(tok, DIM)
    p_scaled = proj_weight * scale

    rows_per_block = 1024
    out = pl.pallas_call(
        _proj_body,
        grid=(tok // rows_per_block,),
        in_specs=[
            pl.BlockSpec((rows_per_block, DIM), lambda i: (i, 0)),
            pl.BlockSpec((MODEL_DIM, DIM), lambda i: (0, 0)),
        ],
        out_specs=pl.BlockSpec((rows_per_block, MODEL_DIM), lambda i: (i, 0)),
        out_shape=jax.ShapeDtypeStruct((tok, MODEL_DIM), jnp.float32),
    )(e, p_scaled)
    return out.reshape(batch, seq_len, MODEL_DIM)


# speedup vs baseline: 1.4456x; 1.4456x over previous
import functools

import jax
import jax.numpy as jnp
from jax import lax
from jax.experimental import pallas as pl
from jax.experimental.pallas import tpu as pltpu
from jax.experimental.pallas import tpu_sc as plsc

VOCAB = 1000000
C = VOCAB - 1
DIM = 64
MODEL_DIM = 1024

NC, NS, LANES = 2, 16, 16
NW = NC * NS

BATCH_T = 64


def _gather_body(seq_len, bpw, tok_hbm, table_hbm, out_hbm, tok_v, h_v,
                 slab_v, sel_v, sem):
    wid = lax.axis_index("s") * NC + lax.axis_index("c")
    base = wid * bpw
    pltpu.sync_copy(tok_hbm.at[pl.ds(base, bpw + 8)], tok_v)
    lane = lax.iota(jnp.int32, LANES)
    for j in range(bpw // LANES):
        cur = tok_v[pl.ds(8 + LANES * j, LANES)]
        prev = tok_v[pl.ds(7 + LANES * j, LANES)]
        h = jnp.mod(
            jnp.bitwise_xor(jnp.int32(36313) * cur, jnp.int32(27191) * prev),
            jnp.int32(C))
        pos = base + (LANES * j) + lane
        h = jnp.where((pos & (seq_len - 1)) == 0, jnp.int32(C), h)
        h_v[pl.ds(LANES * j, LANES)] = h

    gpb = BATCH_T // LANES

    def fetch(g, _):
        hvec = (h_v[pl.ds(g * LANES, LANES)] >> 3) << 3
        k0 = lax.rem(g, jnp.int32(gpb)) * LANES * 8
        for l in range(LANES):
            rb = pl.multiple_of(hvec[l], 8)
            pltpu.make_async_copy(
                table_hbm.at[pl.ds(rb, 8), :],
                slab_v.at[pl.ds(k0 + l * 8, 8), :], sem).start()
        return 0

    for b in range(bpw // BATCH_T):

        def col(d, _):
            dvec = jnp.full((LANES,), 0, jnp.int32) + d
            for g in range(BATCH_T // LANES):
                t0 = b * BATCH_T + g * LANES
                sub = h_v[pl.ds(t0, LANES)] & 7
                ks8 = jnp.int32(g * LANES * 8) + lane * 8 + sub
                vals = plsc.load_gather(slab_v, [ks8, dvec])
                plsc.store_scatter(sel_v, [dvec, jnp.int32(t0) + lane], vals)
            return 0

        lax.fori_loop(b * gpb, (b + 1) * gpb, fetch, 0)
        pltpu.make_async_copy(
            table_hbm.at[pl.ds(0, 8 * BATCH_T), :], slab_v, sem).wait()
        lax.fori_loop(0, DIM, col, 0)
    pltpu.sync_copy(sel_v, out_hbm.at[:, pl.ds(base, bpw)])


def _proj_body(e_ref, p_ref, o_ref):
    o_ref[...] = lax.dot_general(
        e_ref[...], p_ref[...], (((0,), (1,)), ((), ())),
        preferred_element_type=jnp.float32)


@jax.jit
def kernel(token_ids, embed_weight, proj_weight, scale):
    batch, seq_len = token_ids.shape
    tok = batch * seq_len
    bpw = tok // NW

    tok_pad = jnp.concatenate(
        [jnp.zeros((8,), jnp.int32),
         token_ids.reshape(-1)])

    mesh = plsc.VectorSubcoreMesh(core_axis_name="c", subcore_axis_name="s")
    e_t = pl.kernel(
        functools.partial(_gather_body, seq_len, bpw),
        out_type=jax.ShapeDtypeStruct((DIM, tok), jnp.float32),
        mesh=mesh,
        scratch_types=[
            pltpu.VMEM((bpw + 8,), jnp.int32),
            pltpu.VMEM((bpw,), jnp.int32),
            pltpu.VMEM((8 * BATCH_T, DIM), jnp.float32),
            pltpu.VMEM((DIM, bpw), jnp.float32),
            pltpu.SemaphoreType.DMA,
        ],
        compiler_params=pltpu.CompilerParams(use_tc_tiling_on_sc=True,
                                             needs_layout_passes=False),
    )(tok_pad, embed_weight)

    p_scaled = proj_weight * scale

    cols_per_block = 2048
    out = pl.pallas_call(
        _proj_body,
        grid=(tok // cols_per_block,),
        in_specs=[
            pl.BlockSpec((DIM, cols_per_block), lambda i: (0, i)),
            pl.BlockSpec((MODEL_DIM, DIM), lambda i: (0, 0)),
        ],
        out_specs=pl.BlockSpec((cols_per_block, MODEL_DIM), lambda i: (i, 0)),
        out_shape=jax.ShapeDtypeStruct((tok, MODEL_DIM), jnp.float32),
    )(e_t, p_scaled)
    return out.reshape(batch, seq_len, MODEL_DIM)
